# initial kernel scaffold (unmeasured)
import jax
import jax.numpy as jnp
from jax import lax
from jax.experimental import pallas as pl
from jax.experimental.pallas import tpu as pltpu

N_DEV = 4
M = 1024
N = 1024
M_PER = M // N_DEV


def kernel(x, w_mat):
    def body(x_ref, w_ref, out_ref, p_ref, send_ref, recv_ref,
             send_sems, recv_sems):
        my = lax.axis_index("i")
        left = lax.rem(my + N_DEV - 1, N_DEV)
        right = lax.rem(my + 1, N_DEV)

        barrier_sem = pltpu.get_barrier_semaphore()
        for nbr in (left, right):
            pl.semaphore_signal(
                barrier_sem, inc=1,
                device_id=(nbr,), device_id_type=pl.DeviceIdType.MESH,
            )
        pl.semaphore_wait(barrier_sem, 2)

        p_ref[:, :] = jnp.dot(
            x_ref[:, :], w_ref[:, :], preferred_element_type=jnp.float32
        )

        for s in range(N_DEV - 1):
            send_chunk = lax.rem(my + N_DEV - 1 - s, N_DEV)
            off = send_chunk * M_PER
            val = p_ref[pl.ds(off, M_PER), :]
            if s > 0:
                val = val + recv_ref[s - 1].astype(jnp.float32)
            send_ref[:, :] = val.astype(jnp.bfloat16)
            rdma = pltpu.make_async_remote_copy(
                src_ref=send_ref,
                dst_ref=recv_ref.at[s],
                send_sem=send_sems.at[s],
                recv_sem=recv_sems.at[s],
                device_id=(right,),
                device_id_type=pl.DeviceIdType.MESH,
            )
            rdma.start()
            rdma.wait()

        y = (
            p_ref[pl.ds(my * M_PER, M_PER), :]
            + recv_ref[N_DEV - 2].astype(jnp.float32)
        )
        y = jnp.clip(y, -60.0, 60.0)
        out_ref[:, :] = y * (1.0 / (1.0 + jnp.exp(-y)))

    return pl.pallas_call(
        body,
        out_shape=jax.ShapeDtypeStruct((M_PER, N), jnp.float32),
        in_specs=[
            pl.BlockSpec(memory_space=pltpu.VMEM),
            pl.BlockSpec(memory_space=pltpu.VMEM),
        ],
        out_specs=pl.BlockSpec(memory_space=pltpu.VMEM),
        scratch_shapes=[
            pltpu.VMEM((M, N), jnp.float32),
            pltpu.VMEM((M_PER, N), jnp.bfloat16),
            pltpu.VMEM((N_DEV - 1, M_PER, N), jnp.bfloat16),
            pltpu.SemaphoreType.DMA((N_DEV - 1,)),
            pltpu.SemaphoreType.DMA((N_DEV - 1,)),
        ],
        compiler_params=pltpu.CompilerParams(collective_id=0),
    )(x, w_mat)


# baseline (device time: 30015 ns/iter reference)
import jax
import jax.numpy as jnp
from jax import lax
from jax.experimental import pallas as pl
from jax.experimental.pallas import tpu as pltpu

N_DEV = 4
M = 1024
N = 1024
M_PER = M // N_DEV


def kernel(x, w_mat):
    def body(x_ref, w_ref, out_ref, p_ref, send_ref, recv_ref,
             send_sems, recv_sems):
        my = lax.axis_index("i")
        left = lax.rem(my + N_DEV - 1, N_DEV)
        right = lax.rem(my + 1, N_DEV)

        barrier_sem = pltpu.get_barrier_semaphore()
        for nbr in (left, right):
            pl.semaphore_signal(
                barrier_sem, inc=1,
                device_id=(nbr,), device_id_type=pl.DeviceIdType.MESH,
            )
        pl.semaphore_wait(barrier_sem, 2)

        p_ref[:, :] = jnp.dot(
            x_ref[:, :], w_ref[:, :], preferred_element_type=jnp.float32
        )

        for s in range(N_DEV - 1):
            send_chunk = lax.rem(my + N_DEV - 1 - s, N_DEV)
            off = send_chunk * M_PER
            val = p_ref[pl.ds(off, M_PER), :]
            if s > 0:
                val = val + recv_ref[s - 1].astype(jnp.float32)
            send_ref[:, :] = val.astype(jnp.bfloat16)
            rdma = pltpu.make_async_remote_copy(
                src_ref=send_ref,
                dst_ref=recv_ref.at[s],
                send_sem=send_sems.at[s],
                recv_sem=recv_sems.at[s],
                device_id=(right,),
                device_id_type=pl.DeviceIdType.MESH,
            )
            rdma.start()
            rdma.wait()

        y = (
            p_ref[pl.ds(my * M_PER, M_PER), :]
            + recv_ref[N_DEV - 2].astype(jnp.float32)
        )
        yc = jnp.clip(y, -60.0, 60.0)
        out_ref[:, :] = y * (1.0 / (1.0 + jnp.exp(-yc)))

    return pl.pallas_call(
        body,
        out_shape=jax.ShapeDtypeStruct((M_PER, N), jnp.float32),
        in_specs=[
            pl.BlockSpec(memory_space=pltpu.VMEM),
            pl.BlockSpec(memory_space=pltpu.VMEM),
        ],
        out_specs=pl.BlockSpec(memory_space=pltpu.VMEM),
        scratch_shapes=[
            pltpu.VMEM((M, N), jnp.float32),
            pltpu.VMEM((M_PER, N), jnp.bfloat16),
            pltpu.VMEM((N_DEV - 1, M_PER, N), jnp.bfloat16),
            pltpu.SemaphoreType.DMA((N_DEV - 1,)),
            pltpu.SemaphoreType.DMA((N_DEV - 1,)),
        ],
        compiler_params=pltpu.CompilerParams(collective_id=0),
    )(x, w_mat)


# device time: 18284 ns/iter; 1.6416x vs baseline; 1.6416x over previous
import jax
import jax.numpy as jnp
from jax import lax
from jax.experimental import pallas as pl
from jax.experimental.pallas import tpu as pltpu

N_DEV = 4
M = 1024
N = 1024
M_PER = M // N_DEV
H = N // 2


def _silu(y):
    yc = jnp.clip(y, -60.0, 60.0)
    return y * (1.0 / (1.0 + jnp.exp(-yc)))


def kernel(x, w_mat):
    def body(x_ref, w_ref, out_ref, p_ref, so_ref, ri_ref, ssem, rsem):
        my = lax.axis_index("i")
        right = lax.rem(my + 1, N_DEV)
        left = lax.rem(my + N_DEV - 1, N_DEV)

        barrier_sem = pltpu.get_barrier_semaphore()
        for nbr in (left, right):
            pl.semaphore_signal(
                barrier_sem, inc=1,
                device_id=(nbr,), device_id_type=pl.DeviceIdType.MESH,
            )
        pl.semaphore_wait(barrier_sem, 2)

        p_ref[:, :] = jnp.dot(
            x_ref[:, :], w_ref[:, :], preferred_element_type=jnp.float32
        )

        c0 = my * M_PER
        c1 = lax.rem(my + 1, N_DEV) * M_PER
        c2 = lax.rem(my + 2, N_DEV) * M_PER
        c3 = lax.rem(my + 3, N_DEV) * M_PER
        R = pl.ds(0, H)
        L = pl.ds(H, H)

        def mk(slot, dev):
            return pltpu.make_async_remote_copy(
                src_ref=so_ref.at[slot],
                dst_ref=ri_ref.at[slot],
                send_sem=ssem.at[slot],
                recv_sem=rsem.at[slot],
                device_id=(dev,),
                device_id_type=pl.DeviceIdType.MESH,
            )

        so_ref[0] = p_ref[pl.ds(c2, M_PER), R].astype(jnp.bfloat16)
        so_ref[1] = p_ref[pl.ds(c2, M_PER), L].astype(jnp.bfloat16)
        r0 = mk(0, right)
        r1 = mk(1, left)
        r0.start()
        r1.start()

        so_ref[2] = p_ref[pl.ds(c1, M_PER), L].astype(jnp.bfloat16)
        so_ref[3] = p_ref[pl.ds(c3, M_PER), R].astype(jnp.bfloat16)
        r2 = mk(2, right)
        r3 = mk(3, left)
        r2.start()
        r3.start()

        r0.wait_recv()
        so_ref[4] = (
            p_ref[pl.ds(c1, M_PER), R] + ri_ref[0].astype(jnp.float32)
        ).astype(jnp.bfloat16)
        r4 = mk(4, right)
        r4.start()

        r1.wait_recv()
        so_ref[5] = (
            p_ref[pl.ds(c3, M_PER), L] + ri_ref[1].astype(jnp.float32)
        ).astype(jnp.bfloat16)
        r5 = mk(5, left)
        r5.start()

        r3.wait_recv()
        r4.wait_recv()
        y_r = (
            p_ref[pl.ds(c0, M_PER), R]
            + ri_ref[3].astype(jnp.float32)
            + ri_ref[4].astype(jnp.float32)
        )
        out_ref[:, R] = _silu(y_r)

        r2.wait_recv()
        r5.wait_recv()
        y_l = (
            p_ref[pl.ds(c0, M_PER), L]
            + ri_ref[2].astype(jnp.float32)
            + ri_ref[5].astype(jnp.float32)
        )
        out_ref[:, L] = _silu(y_l)

        for r in (r0, r1, r2, r3, r4, r5):
            r.wait_send()

    return pl.pallas_call(
        body,
        out_shape=jax.ShapeDtypeStruct((M_PER, N), jnp.float32),
        in_specs=[
            pl.BlockSpec(memory_space=pltpu.VMEM),
            pl.BlockSpec(memory_space=pltpu.VMEM),
        ],
        out_specs=pl.BlockSpec(memory_space=pltpu.VMEM),
        scratch_shapes=[
            pltpu.VMEM((M, N), jnp.float32),
            pltpu.VMEM((6, M_PER, H), jnp.bfloat16),
            pltpu.VMEM((6, M_PER, H), jnp.bfloat16),
            pltpu.SemaphoreType.DMA((6,)),
            pltpu.SemaphoreType.DMA((6,)),
        ],
        compiler_params=pltpu.CompilerParams(collective_id=0),
    )(x, w_mat)


# device time: 17588 ns/iter; 1.7066x vs baseline; 1.0396x over previous
import jax
import jax.numpy as jnp
from jax import lax
from jax.experimental import pallas as pl
from jax.experimental.pallas import tpu as pltpu

N_DEV = 4
M = 1024
N = 1024
M_PER = M // N_DEV
H = N // 2


def _silu(y):
    yc = jnp.clip(y, -60.0, 60.0)
    return y * (1.0 / (1.0 + jnp.exp(-yc)))


def kernel(x, w_mat):
    def body(x_ref, w_ref, out_ref, p_ref, so_ref, ri_ref, ssem, rsem):
        my = lax.axis_index("i")
        right = lax.rem(my + 1, N_DEV)
        left = lax.rem(my + N_DEV - 1, N_DEV)

        barrier_sem = pltpu.get_barrier_semaphore()
        for nbr in (left, right):
            pl.semaphore_signal(
                barrier_sem, inc=1,
                device_id=(nbr,), device_id_type=pl.DeviceIdType.MESH,
            )
        pl.semaphore_wait(barrier_sem, 2)

        c0 = my * M_PER
        c1 = lax.rem(my + 1, N_DEV) * M_PER
        c2 = lax.rem(my + 2, N_DEV) * M_PER
        c3 = lax.rem(my + 3, N_DEV) * M_PER
        R = pl.ds(0, H)
        L = pl.ds(H, H)

        def chunk_dot(c):
            p_ref[pl.ds(c, M_PER), :] = jnp.dot(
                x_ref[pl.ds(c, M_PER), :], w_ref[:, :],
                preferred_element_type=jnp.float32,
            )

        def mk(slot, dev):
            return pltpu.make_async_remote_copy(
                src_ref=so_ref.at[slot],
                dst_ref=ri_ref.at[slot],
                send_sem=ssem.at[slot],
                recv_sem=rsem.at[slot],
                device_id=(dev,),
                device_id_type=pl.DeviceIdType.MESH,
            )

        chunk_dot(c2)
        so_ref[0] = p_ref[pl.ds(c2, M_PER), R].astype(jnp.bfloat16)
        so_ref[1] = p_ref[pl.ds(c2, M_PER), L].astype(jnp.bfloat16)
        r0 = mk(0, right)
        r1 = mk(1, left)
        r0.start()
        r1.start()

        chunk_dot(c1)
        so_ref[2] = p_ref[pl.ds(c1, M_PER), L].astype(jnp.bfloat16)
        r2 = mk(2, right)
        r2.start()
        chunk_dot(c3)
        so_ref[3] = p_ref[pl.ds(c3, M_PER), R].astype(jnp.bfloat16)
        r3 = mk(3, left)
        r3.start()

        chunk_dot(c0)

        r0.wait_recv()
        so_ref[4] = (
            p_ref[pl.ds(c1, M_PER), R] + ri_ref[0].astype(jnp.float32)
        ).astype(jnp.bfloat16)
        r4 = mk(4, right)
        r4.start()

        r1.wait_recv()
        so_ref[5] = (
            p_ref[pl.ds(c3, M_PER), L] + ri_ref[1].astype(jnp.float32)
        ).astype(jnp.bfloat16)
        r5 = mk(5, left)
        r5.start()

        r3.wait_recv()
        r4.wait_recv()
        y_r = (
            p_ref[pl.ds(c0, M_PER), R]
            + ri_ref[3].astype(jnp.float32)
            + ri_ref[4].astype(jnp.float32)
        )
        out_ref[:, R] = _silu(y_r)

        r2.wait_recv()
        r5.wait_recv()
        y_l = (
            p_ref[pl.ds(c0, M_PER), L]
            + ri_ref[2].astype(jnp.float32)
            + ri_ref[5].astype(jnp.float32)
        )
        out_ref[:, L] = _silu(y_l)

        for r in (r0, r1, r2, r3, r4, r5):
            r.wait_send()

    return pl.pallas_call(
        body,
        out_shape=jax.ShapeDtypeStruct((M_PER, N), jnp.float32),
        in_specs=[
            pl.BlockSpec(memory_space=pltpu.VMEM),
            pl.BlockSpec(memory_space=pltpu.VMEM),
        ],
        out_specs=pl.BlockSpec(memory_space=pltpu.VMEM),
        scratch_shapes=[
            pltpu.VMEM((M, N), jnp.float32),
            pltpu.VMEM((6, M_PER, H), jnp.bfloat16),
            pltpu.VMEM((6, M_PER, H), jnp.bfloat16),
            pltpu.SemaphoreType.DMA((6,)),
            pltpu.SemaphoreType.DMA((6,)),
        ],
        compiler_params=pltpu.CompilerParams(collective_id=0),
    )(x, w_mat)
